# Initial kernel scaffold; baseline (speedup 1.0000x reference)
import jax
import jax.numpy as jnp
from jax.experimental import pallas as pl


def kernel(x, edge_index, edge_attr, node_attr, W_in, b_in, W_out, b_out):
    raise NotImplementedError("write your pallas kernel here")


# speedup vs baseline: 4.3534x; 4.3534x over previous
import functools

import jax
import jax.numpy as jnp
import numpy as np
from jax import lax
from jax.experimental import pallas as pl
from jax.experimental.pallas import tpu as pltpu
from jax.experimental.pallas import tpu_sc as plsc

N = 10000
E = 320000
D = 128
IN_CH = 129
HID = 3
OUT = 128
DP = 136
KP = DP * HID

NC, NS = 2, 16
NW = NC * NS
C = 80
NCHUNK = E // C
CPW = NCHUNK // NW
NBUF = 5
NP = 10240
ZROWS = 128
SPW = NP // NS

_sc_mesh = plsc.VectorSubcoreMesh(
    core_axis_name="c", subcore_axis_name="s", num_cores=NC, num_subcores=NS)


@functools.partial(
    pl.kernel,
    out_type=(jax.ShapeDtypeStruct((E, D), jnp.float32),
              jax.ShapeDtypeStruct((E,), jnp.float32)),
    mesh=_sc_mesh,
    scratch_types=[
        pltpu.VMEM((NBUF, C), jnp.int32),
        pltpu.VMEM((NBUF, C, D), jnp.float32),
        pltpu.VMEM((NBUF, C), jnp.float32),
        pltpu.VMEM((N,), jnp.float32),
        pltpu.SemaphoreType.DMA,
    ],
    compiler_params=pltpu.CompilerParams(needs_layout_passes=False),
)
def _sc_gather(x_hbm, src_hbm, na_hbm, xj_hbm, naj_hbm,
               idx_v, rows_v, nab_v, nat_v, sem):
    wid = lax.axis_index("s") * NC + lax.axis_index("c")
    base = wid * CPW
    pltpu.sync_copy(na_hbm, nat_v)

    def step(j, carry):
        ch = base + j * NBUF
        for b in range(NBUF):
            pltpu.sync_copy(src_hbm.at[pl.ds((ch + b) * C, C)], idx_v.at[b])
        copies = [
            pltpu.async_copy(x_hbm.at[idx_v.at[b]], rows_v.at[b], sem)
            for b in range(NBUF)
        ]
        for b in range(NBUF):
            for k in range(C // 16):
                idx16 = idx_v[b, pl.ds(k * 16, 16)]
                nab_v[b, pl.ds(k * 16, 16)] = plsc.load_gather(nat_v, [idx16])
        for cp in copies:
            cp.wait()
        for b in range(NBUF):
            pltpu.sync_copy(rows_v.at[b],
                            xj_hbm.at[pl.ds((ch + b) * C, C)])
            pltpu.sync_copy(nab_v.at[b],
                            naj_hbm.at[pl.ds((ch + b) * C, C)])
        return carry

    lax.fori_loop(0, CPW // NBUF, step, 0)


SBUF = 3


@functools.partial(
    pl.kernel,
    out_type=jax.ShapeDtypeStruct((NC, NP, OUT), jnp.float32),
    mesh=_sc_mesh,
    scratch_types=[
        pltpu.VMEM((SBUF, C), jnp.int32),
        pltpu.VMEM((SBUF, C, OUT), jnp.float32),
        pltpu.VMEM_SHARED((NP, OUT), jnp.float32),
        pltpu.SemaphoreType.DMA,
    ],
)
def _sc_scatter(proj_hbm, dst_hbm, out_hbm, idx_v, rows_v, acc, sem):
    cid = lax.axis_index("c")
    sid = lax.axis_index("s")
    wid = sid * NC + cid
    base = wid * CPW

    def zstep(t, carry):
        i = t // (OUT // 16)
        j = t % (OUT // 16)
        rows_v[0, i, pl.ds(j * 16, 16)] = jnp.zeros((16,), jnp.float32)
        return carry

    lax.fori_loop(0, C * (OUT // 16), zstep, 0)
    for t in range(SPW // C):
        pltpu.sync_copy(rows_v.at[0],
                        acc.at[pl.ds(sid * SPW + t * C, C)])
    plsc.subcore_barrier()

    def do_chunks(ch, nb):
        for b in range(nb):
            pltpu.sync_copy(dst_hbm.at[pl.ds((ch + b) * C, C)], idx_v.at[b])
        copies = [
            pltpu.async_copy(proj_hbm.at[pl.ds((ch + b) * C, C)],
                             rows_v.at[b], sem)
            for b in range(nb)
        ]
        for cp in copies:
            cp.wait()
        for b in range(nb):
            pltpu.sync_copy(rows_v.at[b], acc.at[idx_v.at[b]], add=True)

    def step(j, carry):
        do_chunks(base + j * SBUF, SBUF)
        return carry

    lax.fori_loop(0, CPW // SBUF, step, 0)
    do_chunks(base + (CPW // SBUF) * SBUF, CPW % SBUF)
    plsc.subcore_barrier()

    pltpu.sync_copy(acc.at[pl.ds(sid * SPW, SPW)],
                    out_hbm.at[cid, pl.ds(sid * SPW, SPW)])


TE = 1600


def _tc_mid_body(ea_ref, xj_ref, na_ref, wi_ref, bi_ref, wo_ref, out_ref):
    scal = jnp.maximum(
        jnp.dot(ea_ref[...], wi_ref[...], preferred_element_type=jnp.float32)
        + bi_ref[...], 0.0)
    t = jnp.concatenate([xj_ref[...], na_ref[...]], axis=1)
    msg = scal * jnp.concatenate([t, t, t], axis=1)
    out_ref[...] = jnp.dot(msg, wo_ref[...],
                           preferred_element_type=jnp.float32)


def _tc_mid(ea, xj, na8, wi, bi, wo):
    return pl.pallas_call(
        _tc_mid_body,
        grid=(E // TE,),
        in_specs=[
            pl.BlockSpec((TE, 8), lambda i: (i, 0)),
            pl.BlockSpec((TE, D), lambda i: (i, 0)),
            pl.BlockSpec((TE, 8), lambda i: (i, 0)),
            pl.BlockSpec((8, KP), lambda i: (0, 0)),
            pl.BlockSpec((1, KP), lambda i: (0, 0)),
            pl.BlockSpec((KP, OUT), lambda i: (0, 0)),
        ],
        out_specs=pl.BlockSpec((TE, OUT), lambda i: (i, 0)),
        out_shape=jax.ShapeDtypeStruct((E, OUT), jnp.float32),
    )(ea, xj, na8, wi, bi, wo)


TN = 2000


def _tc_final_body(p0_ref, p1_ref, b_ref, out_ref):
    out_ref[...] = jnp.tanh(p0_ref[...] + p1_ref[...] + b_ref[...])


def _tc_final(p0, p1, b):
    return pl.pallas_call(
        _tc_final_body,
        grid=(N // TN,),
        in_specs=[
            pl.BlockSpec((TN, OUT), lambda i: (i, 0)),
            pl.BlockSpec((TN, OUT), lambda i: (i, 0)),
            pl.BlockSpec((1, OUT), lambda i: (0, 0)),
        ],
        out_specs=pl.BlockSpec((TN, OUT), lambda i: (i, 0)),
        out_shape=jax.ShapeDtypeStruct((N, OUT), jnp.float32),
    )(p0, p1, b)


_k2 = np.arange(KP)
_c_of_k2 = _k2 % DP
_h_of_k2 = _k2 // DP
_VALID = _c_of_k2 < IN_CH
_OLDK = np.where(_c_of_k2 < IN_CH, _c_of_k2 * HID + _h_of_k2, 0)


def kernel(x, edge_index, edge_attr, node_attr, W_in, b_in, W_out, b_out):
    src = edge_index[0]
    dst = edge_index[1]
    na = node_attr.reshape(N)
    ea = jnp.concatenate([edge_attr, jnp.zeros((E, 2), jnp.float32)], axis=1)
    wg = jnp.where(_VALID[:, None], W_in[_OLDK], 0.0)
    wi = jnp.concatenate([wg.T, jnp.zeros((2, KP), jnp.float32)], axis=0)
    bi = jnp.where(_VALID, b_in[_OLDK], 0.0).reshape(1, KP)
    wo = jnp.where(_VALID[:, None], W_out.T[_OLDK], 0.0)

    xj, naj = _sc_gather(x, src, na)
    na8 = jnp.pad(naj.reshape(E, 1), ((0, 0), (0, 7)))
    proj = _tc_mid(ea, xj, na8, wi, bi, wo)
    parts = _sc_scatter(proj, dst)
    return _tc_final(parts[0], parts[1], b_out.reshape(1, OUT))


# speedup vs baseline: 6.1860x; 1.4210x over previous
import functools

import jax
import jax.numpy as jnp
import numpy as np
from jax import lax
from jax.experimental import pallas as pl
from jax.experimental.pallas import tpu as pltpu
from jax.experimental.pallas import tpu_sc as plsc

N = 10000
E = 320000
D = 128
IN_CH = 129
HID = 3
OUT = 128
KP2 = 512

NC, NS = 2, 16
NW = NC * NS
C = 80
NCHUNK = E // C
CPW = NCHUNK // NW
NBUF = 5
NP = 10240
ZROWS = 128
SPW = NP // NS

_sc_mesh = plsc.VectorSubcoreMesh(
    core_axis_name="c", subcore_axis_name="s", num_cores=NC, num_subcores=NS)


@functools.partial(
    pl.kernel,
    out_type=(jax.ShapeDtypeStruct((E, D), jnp.float32),
              jax.ShapeDtypeStruct((E,), jnp.float32)),
    mesh=_sc_mesh,
    scratch_types=[
        pltpu.VMEM((NBUF, C), jnp.int32),
        pltpu.VMEM((NBUF, C, D), jnp.float32),
        pltpu.VMEM((NBUF, C), jnp.float32),
        pltpu.VMEM((N,), jnp.float32),
        pltpu.SemaphoreType.DMA,
    ],
    compiler_params=pltpu.CompilerParams(needs_layout_passes=False),
)
def _sc_gather(x_hbm, src_hbm, na_hbm, xj_hbm, naj_hbm,
               idx_v, rows_v, nab_v, nat_v, sem):
    wid = lax.axis_index("s") * NC + lax.axis_index("c")
    base = wid * CPW
    pltpu.sync_copy(na_hbm, nat_v)

    def step(j, carry):
        ch = base + j * NBUF
        for b in range(NBUF):
            pltpu.sync_copy(src_hbm.at[pl.ds((ch + b) * C, C)], idx_v.at[b])
        copies = [
            pltpu.async_copy(x_hbm.at[idx_v.at[b]], rows_v.at[b], sem)
            for b in range(NBUF)
        ]
        for b in range(NBUF):
            for k in range(C // 16):
                idx16 = idx_v[b, pl.ds(k * 16, 16)]
                nab_v[b, pl.ds(k * 16, 16)] = plsc.load_gather(nat_v, [idx16])
        for cp in copies:
            cp.wait()
        for b in range(NBUF):
            pltpu.sync_copy(rows_v.at[b],
                            xj_hbm.at[pl.ds((ch + b) * C, C)])
            pltpu.sync_copy(nab_v.at[b],
                            naj_hbm.at[pl.ds((ch + b) * C, C)])
        return carry

    lax.fori_loop(0, CPW // NBUF, step, 0)


SBUF = 3


@functools.partial(
    pl.kernel,
    out_type=jax.ShapeDtypeStruct((NC, NP, OUT), jnp.float32),
    mesh=_sc_mesh,
    scratch_types=[
        pltpu.VMEM((SBUF, C), jnp.int32),
        pltpu.VMEM((SBUF, C, OUT), jnp.float32),
        pltpu.VMEM_SHARED((NP, OUT), jnp.float32),
        pltpu.SemaphoreType.DMA,
    ],
)
def _sc_scatter(proj_hbm, dst_hbm, out_hbm, idx_v, rows_v, acc, sem):
    cid = lax.axis_index("c")
    sid = lax.axis_index("s")
    wid = sid * NC + cid
    base = wid * CPW

    def zstep(t, carry):
        i = t // (OUT // 16)
        j = t % (OUT // 16)
        rows_v[0, i, pl.ds(j * 16, 16)] = jnp.zeros((16,), jnp.float32)
        return carry

    lax.fori_loop(0, C * (OUT // 16), zstep, 0)
    for t in range(SPW // C):
        pltpu.sync_copy(rows_v.at[0],
                        acc.at[pl.ds(sid * SPW + t * C, C)])
    plsc.subcore_barrier()

    def do_chunks(ch, nb):
        for b in range(nb):
            pltpu.sync_copy(dst_hbm.at[pl.ds((ch + b) * C, C)], idx_v.at[b])
        copies = [
            pltpu.async_copy(proj_hbm.at[pl.ds((ch + b) * C, C)],
                             rows_v.at[b], sem)
            for b in range(nb)
        ]
        for cp in copies:
            cp.wait()
        for b in range(nb):
            pltpu.sync_copy(rows_v.at[b], acc.at[idx_v.at[b]], add=True)

    def step(j, carry):
        do_chunks(base + j * SBUF, SBUF)
        return carry

    lax.fori_loop(0, CPW // SBUF, step, 0)
    do_chunks(base + (CPW // SBUF) * SBUF, CPW % SBUF)
    plsc.subcore_barrier()

    pltpu.sync_copy(acc.at[pl.ds(sid * SPW, SPW)],
                    out_hbm.at[cid, pl.ds(sid * SPW, SPW)])


TE = 2560


def _tc_mid_body(eat_ref, xj_ref, wi_ref, bi_ref, wo_ref, out_ref):
    eat = eat_ref[...]
    scal = jnp.maximum(
        lax.dot_general(eat, wi_ref[...], (((0,), (0,)), ((), ())),
                        preferred_element_type=jnp.float32)
        + bi_ref[...], 0.0)
    sel = (lax.broadcasted_iota(jnp.int32, (8, 1), 0) == 6
           ).astype(jnp.float32)
    na_col = lax.dot_general(eat, sel, (((0,), (0,)), ((), ())),
                             preferred_element_type=jnp.float32)
    xj = xj_ref[...]
    base = jnp.concatenate(
        [xj, xj, xj, jnp.broadcast_to(na_col, (TE, D))], axis=1)
    msg = scal * base
    out_ref[...] = jnp.dot(msg, wo_ref[...],
                           preferred_element_type=jnp.float32)


def _tc_mid(eat, xj, wi, bi, wo):
    return pl.pallas_call(
        _tc_mid_body,
        grid=(E // TE,),
        in_specs=[
            pl.BlockSpec((8, TE), lambda i: (0, i)),
            pl.BlockSpec((TE, D), lambda i: (i, 0)),
            pl.BlockSpec((8, KP2), lambda i: (0, 0)),
            pl.BlockSpec((1, KP2), lambda i: (0, 0)),
            pl.BlockSpec((KP2, OUT), lambda i: (0, 0)),
        ],
        out_specs=pl.BlockSpec((TE, OUT), lambda i: (i, 0)),
        out_shape=jax.ShapeDtypeStruct((E, OUT), jnp.float32),
    )(eat, xj, wi, bi, wo)


TN = 2000


def _tc_final_body(p0_ref, p1_ref, b_ref, out_ref):
    out_ref[...] = jnp.tanh(p0_ref[...] + p1_ref[...] + b_ref[...])


def _tc_final(p0, p1, b):
    return pl.pallas_call(
        _tc_final_body,
        grid=(N // TN,),
        in_specs=[
            pl.BlockSpec((TN, OUT), lambda i: (i, 0)),
            pl.BlockSpec((TN, OUT), lambda i: (i, 0)),
            pl.BlockSpec((1, OUT), lambda i: (0, 0)),
        ],
        out_specs=pl.BlockSpec((TN, OUT), lambda i: (i, 0)),
        out_shape=jax.ShapeDtypeStruct((N, OUT), jnp.float32),
    )(p0, p1, b)


_k2 = np.arange(KP2)
_VALID = _k2 < HID * IN_CH
_OLDK = np.where(_k2 < 384, (_k2 % D) * HID + _k2 // D,
                 np.minimum(_k2, HID * IN_CH - 1))


def kernel(x, edge_index, edge_attr, node_attr, W_in, b_in, W_out, b_out):
    src = edge_index[0]
    dst = edge_index[1]
    na = node_attr.reshape(N)
    wg = jnp.where(_VALID[:, None], W_in[_OLDK], 0.0)
    wi = jnp.concatenate([wg.T, jnp.zeros((2, KP2), jnp.float32)], axis=0)
    bi = jnp.where(_VALID, b_in[_OLDK], 0.0).reshape(1, KP2)
    wo = jnp.where(_VALID[:, None], W_out.T[_OLDK], 0.0)

    xj, naj = _sc_gather(x, src, na)
    eat = jnp.concatenate(
        [edge_attr.T, naj.reshape(1, E), jnp.zeros((1, E), jnp.float32)],
        axis=0)
    proj = _tc_mid(eat, xj, wi, bi, wo)
    parts = _sc_scatter(proj, dst)
    return _tc_final(parts[0], parts[1], b_out.reshape(1, OUT))


# speedup vs baseline: 6.5993x; 1.0668x over previous
import functools

import jax
import jax.numpy as jnp
import numpy as np
from jax import lax
from jax.experimental import pallas as pl
from jax.experimental.pallas import tpu as pltpu
from jax.experimental.pallas import tpu_sc as plsc

N = 10000
E = 320000
D = 128
IN_CH = 129
HID = 3
OUT = 128
KP2 = 512

NC, NS = 2, 16
NW = NC * NS
EPW = E // NW
C = 80
CPW = EPW // C
NBUF = 5
SBUF = 3
NP = 10240
SPW = NP // NS

_sc_mesh = plsc.VectorSubcoreMesh(
    core_axis_name="c", subcore_axis_name="s", num_cores=NC, num_subcores=NS)

_NA_OFFS = list(range(0, C - 15, 16)) + ([C - 16] if C % 16 else [])


@functools.partial(
    pl.kernel,
    out_type=(jax.ShapeDtypeStruct((E, D), jnp.float32),
              jax.ShapeDtypeStruct((E,), jnp.float32)),
    mesh=_sc_mesh,
    scratch_types=[
        pltpu.VMEM((NBUF, C), jnp.int32),
        pltpu.VMEM((NBUF, C, D), jnp.float32),
        pltpu.VMEM((NBUF, C), jnp.float32),
        pltpu.VMEM((N,), jnp.float32),
        pltpu.SemaphoreType.DMA,
    ],
    compiler_params=pltpu.CompilerParams(needs_layout_passes=False),
)
def _sc_gather(x_hbm, src_hbm, na_hbm, xj_hbm, naj_hbm,
               idx_v, rows_v, nab_v, nat_v, sem):
    wid = lax.axis_index("s") * NC + lax.axis_index("c")
    obase = wid * EPW
    pltpu.sync_copy(na_hbm, nat_v)

    def group(j, carry):
        ch = j * NBUF
        for b in range(NBUF):
            pltpu.sync_copy(src_hbm.at[pl.ds(obase + (ch + b) * C, C)],
                            idx_v.at[b])
        copies = [
            pltpu.async_copy(x_hbm.at[idx_v.at[b]], rows_v.at[b], sem)
            for b in range(NBUF)
        ]
        for b in range(NBUF):
            for o in _NA_OFFS:
                nab_v[b, pl.ds(o, 16)] = plsc.load_gather(
                    nat_v, [idx_v[b, pl.ds(o, 16)]])
        for b in range(NBUF):
            copies[b].wait()
            pltpu.sync_copy(rows_v.at[b],
                            xj_hbm.at[pl.ds(obase + (ch + b) * C, C)])
            pltpu.sync_copy(nab_v.at[b],
                            naj_hbm.at[pl.ds(obase + (ch + b) * C, C)])
        return carry

    lax.fori_loop(0, CPW // NBUF, group, 0)


@functools.partial(
    pl.kernel,
    out_type=jax.ShapeDtypeStruct((NC, NP, OUT), jnp.float32),
    mesh=_sc_mesh,
    scratch_types=[
        pltpu.VMEM((SBUF, C), jnp.int32),
        pltpu.VMEM((SBUF, C, OUT), jnp.float32),
        pltpu.VMEM_SHARED((NP, OUT), jnp.float32),
        pltpu.SemaphoreType.DMA,
    ],
)
def _sc_scatter(proj_hbm, dst_hbm, out_hbm, idx_v, rows_v, acc, sem):
    cid = lax.axis_index("c")
    sid = lax.axis_index("s")
    wid = sid * NC + cid
    obase = wid * EPW

    def zstep(t, carry):
        i = t // (OUT // 16)
        j = t % (OUT // 16)
        rows_v[0, i, pl.ds(j * 16, 16)] = jnp.zeros((16,), jnp.float32)
        return carry

    lax.fori_loop(0, C * (OUT // 16), zstep, 0)
    for t in range(SPW // C):
        pltpu.sync_copy(rows_v.at[0], acc.at[pl.ds(sid * SPW + t * C, C)])
    plsc.subcore_barrier()

    def do_chunks(ch, nb):
        for b in range(nb):
            pltpu.sync_copy(dst_hbm.at[pl.ds(obase + (ch + b) * C, C)],
                            idx_v.at[b])
        copies = [
            pltpu.async_copy(proj_hbm.at[pl.ds(obase + (ch + b) * C, C)],
                             rows_v.at[b], sem)
            for b in range(nb)
        ]
        for b in range(nb):
            copies[b].wait()
            pltpu.sync_copy(rows_v.at[b], acc.at[idx_v.at[b]], add=True)

    def group(j, carry):
        do_chunks(j * SBUF, SBUF)
        return carry

    ngroup = CPW // SBUF
    lax.fori_loop(0, ngroup, group, 0)
    do_chunks(ngroup * SBUF, CPW % SBUF)
    plsc.subcore_barrier()

    pltpu.sync_copy(acc.at[pl.ds(sid * SPW, SPW)],
                    out_hbm.at[cid, pl.ds(sid * SPW, SPW)])


TE = 2560


def _tc_mid_body(eat_ref, xj_ref, wi_ref, bi_ref, wo_ref, out_ref):
    eat = eat_ref[...]
    scal = jnp.maximum(
        lax.dot_general(eat, wi_ref[...], (((0,), (0,)), ((), ())),
                        preferred_element_type=jnp.float32)
        + bi_ref[...], 0.0)
    sel = (lax.broadcasted_iota(jnp.int32, (8, 1), 0) == 6
           ).astype(jnp.bfloat16)
    na_col = lax.dot_general(eat, sel, (((0,), (0,)), ((), ())),
                             preferred_element_type=jnp.float32)
    xj = xj_ref[...]
    base = jnp.concatenate(
        [xj, xj, xj, jnp.broadcast_to(na_col, (TE, D))], axis=1)
    msg = (scal * base).astype(jnp.bfloat16)
    out_ref[...] = jnp.dot(msg, wo_ref[...],
                           preferred_element_type=jnp.float32)


def _tc_mid(eat, xj, wi, bi, wo):
    return pl.pallas_call(
        _tc_mid_body,
        grid=(E // TE,),
        in_specs=[
            pl.BlockSpec((8, TE), lambda i: (0, i)),
            pl.BlockSpec((TE, D), lambda i: (i, 0)),
            pl.BlockSpec((8, KP2), lambda i: (0, 0)),
            pl.BlockSpec((1, KP2), lambda i: (0, 0)),
            pl.BlockSpec((KP2, OUT), lambda i: (0, 0)),
        ],
        out_specs=pl.BlockSpec((TE, OUT), lambda i: (i, 0)),
        out_shape=jax.ShapeDtypeStruct((E, OUT), jnp.float32),
    )(eat, xj, wi, bi, wo)


TN = 2000


def _tc_final_body(p0_ref, p1_ref, b_ref, out_ref):
    out_ref[...] = jnp.tanh(p0_ref[...] + p1_ref[...] + b_ref[...])


def _tc_final(p0, p1, b):
    return pl.pallas_call(
        _tc_final_body,
        grid=(N // TN,),
        in_specs=[
            pl.BlockSpec((TN, OUT), lambda i: (i, 0)),
            pl.BlockSpec((TN, OUT), lambda i: (i, 0)),
            pl.BlockSpec((1, OUT), lambda i: (0, 0)),
        ],
        out_specs=pl.BlockSpec((TN, OUT), lambda i: (i, 0)),
        out_shape=jax.ShapeDtypeStruct((N, OUT), jnp.float32),
    )(p0, p1, b)


_k2 = np.arange(KP2)
_VALID = _k2 < HID * IN_CH
_OLDK = np.where(_k2 < 384, (_k2 % D) * HID + _k2 // D,
                 np.minimum(_k2, HID * IN_CH - 1))


def kernel(x, edge_index, edge_attr, node_attr, W_in, b_in, W_out, b_out):
    src = edge_index[0]
    dst = edge_index[1]
    na = node_attr.reshape(N)
    wg = jnp.where(_VALID[:, None], W_in[_OLDK], 0.0)
    wi = jnp.concatenate(
        [wg.T, jnp.zeros((2, KP2), jnp.float32)], axis=0).astype(jnp.bfloat16)
    bi = jnp.where(_VALID, b_in[_OLDK], 0.0).reshape(1, KP2)
    wo = jnp.where(_VALID[:, None], W_out.T[_OLDK],
                   0.0).astype(jnp.bfloat16)

    xj, naj = _sc_gather(x, src, na)
    eat = jnp.concatenate(
        [edge_attr.T, naj.reshape(1, E), jnp.zeros((1, E), jnp.float32)],
        axis=0).astype(jnp.bfloat16)
    proj = _tc_mid(eat, xj, wi, bi, wo)
    parts = _sc_scatter(proj, dst)
    return _tc_final(parts[0], parts[1], b_out.reshape(1, OUT))


# speedup vs baseline: 7.0809x; 1.0730x over previous
import functools

import jax
import jax.numpy as jnp
import numpy as np
from jax import lax
from jax.experimental import pallas as pl
from jax.experimental.pallas import tpu as pltpu
from jax.experimental.pallas import tpu_sc as plsc

N = 10000
E = 320000
D = 128
IN_CH = 129
HID = 3
OUT = 128
KP2 = 512

NC, NS = 2, 16
NW = NC * NS
EPW = E // NW
C = 80
CPW = EPW // C
NBUF = 5
SBUF = 3
NP = 10240
SPW = NP // NS

_sc_mesh = plsc.VectorSubcoreMesh(
    core_axis_name="c", subcore_axis_name="s", num_cores=NC, num_subcores=NS)

_NA_OFFS = list(range(0, C - 15, 16)) + ([C - 16] if C % 16 else [])


@functools.partial(
    pl.kernel,
    out_type=(jax.ShapeDtypeStruct((E, D), jnp.float32),
              jax.ShapeDtypeStruct((E,), jnp.float32)),
    mesh=_sc_mesh,
    scratch_types=[
        pltpu.VMEM((NBUF, C), jnp.int32),
        pltpu.VMEM((NBUF, C, D), jnp.float32),
        pltpu.VMEM((NBUF, C), jnp.float32),
        pltpu.VMEM((N,), jnp.float32),
        pltpu.SemaphoreType.DMA,
        pltpu.SemaphoreType.DMA,
        pltpu.SemaphoreType.DMA((NBUF,)),
    ],
    compiler_params=pltpu.CompilerParams(needs_layout_passes=False),
)
def _sc_gather(x_hbm, src_hbm, na_hbm, xj_hbm, naj_hbm,
               idx_v, rows_v, nab_v, nat_v, sem_i, sem_g, sem_w):
    wid = lax.axis_index("s") * NC + lax.axis_index("c")
    obase = wid * EPW
    pltpu.sync_copy(na_hbm, nat_v)

    def wr_x(j, b):
        return pltpu.make_async_copy(
            rows_v.at[b],
            xj_hbm.at[pl.ds(obase + (j * NBUF + b) * C, C)], sem_w.at[b])

    def wr_na(j, b):
        return pltpu.make_async_copy(
            nab_v.at[b],
            naj_hbm.at[pl.ds(obase + (j * NBUF + b) * C, C)], sem_w.at[b])

    def group(j, carry):
        ch = j * NBUF
        idx_c = [
            pltpu.async_copy(src_hbm.at[pl.ds(obase + (ch + b) * C, C)],
                             idx_v.at[b], sem_i)
            for b in range(NBUF)
        ]
        for b in range(NBUF):
            @pl.when(j > 0)
            def _():
                wr_x(j - 1, b).wait()
                wr_na(j - 1, b).wait()
        copies = []
        for b in range(NBUF):
            idx_c[b].wait()
            copies.append(
                pltpu.async_copy(x_hbm.at[idx_v.at[b]], rows_v.at[b], sem_g))
        for b in range(NBUF):
            for o in _NA_OFFS:
                nab_v[b, pl.ds(o, 16)] = plsc.load_gather(
                    nat_v, [idx_v[b, pl.ds(o, 16)]])
        for b in range(NBUF):
            copies[b].wait()
            wr_x(j, b).start()
            wr_na(j, b).start()
        return carry

    lax.fori_loop(0, CPW // NBUF, group, 0)
    for b in range(NBUF):
        wr_x(CPW // NBUF - 1, b).wait()
        wr_na(CPW // NBUF - 1, b).wait()


@functools.partial(
    pl.kernel,
    out_type=jax.ShapeDtypeStruct((NC, NP, OUT), jnp.float32),
    mesh=_sc_mesh,
    scratch_types=[
        pltpu.VMEM((SBUF, C), jnp.int32),
        pltpu.VMEM((SBUF, C, OUT), jnp.float32),
        pltpu.VMEM_SHARED((NP, OUT), jnp.float32),
        pltpu.SemaphoreType.DMA,
    ],
)
def _sc_scatter(proj_hbm, dst_hbm, out_hbm, idx_v, rows_v, acc, sem):
    cid = lax.axis_index("c")
    sid = lax.axis_index("s")
    wid = sid * NC + cid
    obase = wid * EPW

    def zstep(t, carry):
        i = t // (OUT // 16)
        j = t % (OUT // 16)
        rows_v[0, i, pl.ds(j * 16, 16)] = jnp.zeros((16,), jnp.float32)
        return carry

    lax.fori_loop(0, C * (OUT // 16), zstep, 0)
    for t in range(SPW // C):
        pltpu.sync_copy(rows_v.at[0], acc.at[pl.ds(sid * SPW + t * C, C)])
    plsc.subcore_barrier()

    def do_chunks(ch, nb):
        for b in range(nb):
            pltpu.sync_copy(dst_hbm.at[pl.ds(obase + (ch + b) * C, C)],
                            idx_v.at[b])
        copies = [
            pltpu.async_copy(proj_hbm.at[pl.ds(obase + (ch + b) * C, C)],
                             rows_v.at[b], sem)
            for b in range(nb)
        ]
        for b in range(nb):
            copies[b].wait()
            pltpu.sync_copy(rows_v.at[b], acc.at[idx_v.at[b]], add=True)

    def group(j, carry):
        do_chunks(j * SBUF, SBUF)
        return carry

    ngroup = CPW // SBUF
    lax.fori_loop(0, ngroup, group, 0)
    do_chunks(ngroup * SBUF, CPW % SBUF)
    plsc.subcore_barrier()

    pltpu.sync_copy(acc.at[pl.ds(sid * SPW, SPW)],
                    out_hbm.at[cid, pl.ds(sid * SPW, SPW)])


TE = 2560


def _tc_mid_body(eat_ref, xj_ref, wi_ref, bi_ref, wo_ref, out_ref):
    eat = eat_ref[...]
    scal = jnp.maximum(
        lax.dot_general(eat, wi_ref[...], (((0,), (0,)), ((), ())),
                        preferred_element_type=jnp.float32)
        + bi_ref[...], 0.0)
    sel = (lax.broadcasted_iota(jnp.int32, (8, 1), 0) == 6
           ).astype(jnp.bfloat16)
    na_col = lax.dot_general(eat, sel, (((0,), (0,)), ((), ())),
                             preferred_element_type=jnp.float32)
    xj = xj_ref[...]
    base = jnp.concatenate(
        [xj, xj, xj, jnp.broadcast_to(na_col, (TE, D))], axis=1)
    msg = (scal * base).astype(jnp.bfloat16)
    out_ref[...] = jnp.dot(msg, wo_ref[...],
                           preferred_element_type=jnp.float32)


def _tc_mid(eat, xj, wi, bi, wo):
    return pl.pallas_call(
        _tc_mid_body,
        grid=(E // TE,),
        in_specs=[
            pl.BlockSpec((8, TE), lambda i: (0, i)),
            pl.BlockSpec((TE, D), lambda i: (i, 0)),
            pl.BlockSpec((8, KP2), lambda i: (0, 0)),
            pl.BlockSpec((1, KP2), lambda i: (0, 0)),
            pl.BlockSpec((KP2, OUT), lambda i: (0, 0)),
        ],
        out_specs=pl.BlockSpec((TE, OUT), lambda i: (i, 0)),
        out_shape=jax.ShapeDtypeStruct((E, OUT), jnp.float32),
    )(eat, xj, wi, bi, wo)


TN = 2000


def _tc_final_body(p0_ref, p1_ref, b_ref, out_ref):
    out_ref[...] = jnp.tanh(p0_ref[...] + p1_ref[...] + b_ref[...])


def _tc_final(p0, p1, b):
    return pl.pallas_call(
        _tc_final_body,
        grid=(N // TN,),
        in_specs=[
            pl.BlockSpec((TN, OUT), lambda i: (i, 0)),
            pl.BlockSpec((TN, OUT), lambda i: (i, 0)),
            pl.BlockSpec((1, OUT), lambda i: (0, 0)),
        ],
        out_specs=pl.BlockSpec((TN, OUT), lambda i: (i, 0)),
        out_shape=jax.ShapeDtypeStruct((N, OUT), jnp.float32),
    )(p0, p1, b)


_k2 = np.arange(KP2)
_VALID = _k2 < HID * IN_CH
_OLDK = np.where(_k2 < 384, (_k2 % D) * HID + _k2 // D,
                 np.minimum(_k2, HID * IN_CH - 1))


def kernel(x, edge_index, edge_attr, node_attr, W_in, b_in, W_out, b_out):
    src = edge_index[0]
    dst = edge_index[1]
    na = node_attr.reshape(N)
    wg = jnp.where(_VALID[:, None], W_in[_OLDK], 0.0)
    wi = jnp.concatenate(
        [wg.T, jnp.zeros((2, KP2), jnp.float32)], axis=0).astype(jnp.bfloat16)
    bi = jnp.where(_VALID, b_in[_OLDK], 0.0).reshape(1, KP2)
    wo = jnp.where(_VALID[:, None], W_out.T[_OLDK],
                   0.0).astype(jnp.bfloat16)

    xj, naj = _sc_gather(x, src, na)
    eat = jnp.concatenate(
        [edge_attr.T, naj.reshape(1, E), jnp.zeros((1, E), jnp.float32)],
        axis=0).astype(jnp.bfloat16)
    proj = _tc_mid(eat, xj, wi, bi, wo)
    parts = _sc_scatter(proj, dst)
    return _tc_final(parts[0], parts[1], b_out.reshape(1, OUT))


# speedup vs baseline: 7.6794x; 1.0845x over previous
import functools

import jax
import jax.numpy as jnp
import numpy as np
from jax import lax
from jax.experimental import pallas as pl
from jax.experimental.pallas import tpu as pltpu
from jax.experimental.pallas import tpu_sc as plsc

N = 10000
E = 320000
D = 128
IN_CH = 129
HID = 3
OUT = 128
KP2 = 512

NC, NS = 2, 16
NW = NC * NS
EPW = E // NW
C = 80
CPW = EPW // C
NBUF = 5
SBUF = 3
NP = 10240
SPW = NP // NS

_sc_mesh = plsc.VectorSubcoreMesh(
    core_axis_name="c", subcore_axis_name="s", num_cores=NC, num_subcores=NS)

_NA_OFFS = list(range(0, C - 15, 16)) + ([C - 16] if C % 16 else [])


@functools.partial(
    pl.kernel,
    out_type=(jax.ShapeDtypeStruct((E, D), jnp.float32),
              jax.ShapeDtypeStruct((E,), jnp.float32)),
    mesh=_sc_mesh,
    scratch_types=[
        pltpu.VMEM((NBUF, C), jnp.int32),
        pltpu.VMEM((NBUF, C, D), jnp.float32),
        pltpu.VMEM((NBUF, C), jnp.float32),
        pltpu.VMEM((N,), jnp.float32),
        pltpu.SemaphoreType.DMA,
        pltpu.SemaphoreType.DMA,
        pltpu.SemaphoreType.DMA((NBUF,)),
    ],
    compiler_params=pltpu.CompilerParams(needs_layout_passes=False),
)
def _sc_gather(x_hbm, src_hbm, na_hbm, xj_hbm, naj_hbm,
               idx_v, rows_v, nab_v, nat_v, sem_i, sem_g, sem_w):
    wid = lax.axis_index("s") * NC + lax.axis_index("c")
    obase = wid * EPW
    pltpu.sync_copy(na_hbm, nat_v)

    def wr_x(j, b):
        return pltpu.make_async_copy(
            rows_v.at[b],
            xj_hbm.at[pl.ds(obase + (j * NBUF + b) * C, C)], sem_w.at[b])

    def wr_na(j, b):
        return pltpu.make_async_copy(
            nab_v.at[b],
            naj_hbm.at[pl.ds(obase + (j * NBUF + b) * C, C)], sem_w.at[b])

    def group(j, carry):
        ch = j * NBUF
        idx_c = [
            pltpu.async_copy(src_hbm.at[pl.ds(obase + (ch + b) * C, C)],
                             idx_v.at[b], sem_i)
            for b in range(NBUF)
        ]
        for b in range(NBUF):
            @pl.when(j > 0)
            def _():
                wr_x(j - 1, b).wait()
                wr_na(j - 1, b).wait()
        copies = []
        for b in range(NBUF):
            idx_c[b].wait()
            copies.append(
                pltpu.async_copy(x_hbm.at[idx_v.at[b]], rows_v.at[b], sem_g))
        for b in range(NBUF):
            for o in _NA_OFFS:
                nab_v[b, pl.ds(o, 16)] = plsc.load_gather(
                    nat_v, [idx_v[b, pl.ds(o, 16)]])
        for b in range(NBUF):
            copies[b].wait()
            wr_x(j, b).start()
            wr_na(j, b).start()
        return carry

    lax.fori_loop(0, CPW // NBUF, group, 0)
    for b in range(NBUF):
        wr_x(CPW // NBUF - 1, b).wait()
        wr_na(CPW // NBUF - 1, b).wait()


@functools.partial(
    pl.kernel,
    out_type=jax.ShapeDtypeStruct((NC, NP, OUT), jnp.float32),
    mesh=_sc_mesh,
    scratch_types=[
        pltpu.VMEM((SBUF, C), jnp.int32),
        pltpu.VMEM((SBUF, C, OUT), jnp.float32),
        pltpu.VMEM_SHARED((NP, OUT), jnp.float32),
        pltpu.SemaphoreType.DMA,
        pltpu.SemaphoreType.DMA,
        pltpu.SemaphoreType.DMA((SBUF,)),
    ],
)
def _sc_scatter(proj_hbm, dst_hbm, out_hbm, idx_v, rows_v, acc,
                sem_i, sem_l, sem_a):
    cid = lax.axis_index("c")
    sid = lax.axis_index("s")
    wid = sid * NC + cid
    obase = wid * EPW

    def zstep(t, carry):
        i = t // (OUT // 16)
        j = t % (OUT // 16)
        rows_v[0, i, pl.ds(j * 16, 16)] = jnp.zeros((16,), jnp.float32)
        return carry

    lax.fori_loop(0, C * (OUT // 16), zstep, 0)
    for t in range(SPW // C):
        pltpu.sync_copy(rows_v.at[0], acc.at[pl.ds(sid * SPW + t * C, C)])
    plsc.subcore_barrier()

    def add_start(b):
        pltpu.async_copy(rows_v.at[b], acc.at[idx_v.at[b]],
                         sem_a.at[b], add=True)

    def add_wait(b):
        pltpu.make_async_copy(rows_v.at[b], acc.at[idx_v.at[b]],
                              sem_a.at[b]).wait()

    def group(j, carry):
        ch = j * SBUF
        for b in range(SBUF):
            @pl.when(j > 0)
            def _():
                add_wait(b)
        idx_c = [
            pltpu.async_copy(dst_hbm.at[pl.ds(obase + (ch + b) * C, C)],
                             idx_v.at[b], sem_i)
            for b in range(SBUF)
        ]
        copies = [
            pltpu.async_copy(proj_hbm.at[pl.ds(obase + (ch + b) * C, C)],
                             rows_v.at[b], sem_l)
            for b in range(SBUF)
        ]
        for b in range(SBUF):
            idx_c[b].wait()
            copies[b].wait()
            add_start(b)
        return carry

    ngroup = CPW // SBUF
    lax.fori_loop(0, ngroup, group, 0)
    for b in range(SBUF):
        add_wait(b)
    for r in range(CPW % SBUF):
        lb = ngroup * SBUF + r
        pltpu.sync_copy(dst_hbm.at[pl.ds(obase + lb * C, C)], idx_v.at[r])
        pltpu.async_copy(proj_hbm.at[pl.ds(obase + lb * C, C)],
                         rows_v.at[r], sem_l).wait()
        pltpu.sync_copy(rows_v.at[r], acc.at[idx_v.at[r]], add=True)
    plsc.subcore_barrier()

    pltpu.sync_copy(acc.at[pl.ds(sid * SPW, SPW)],
                    out_hbm.at[cid, pl.ds(sid * SPW, SPW)])


TE = 2560


def _tc_mid_body(eat_ref, xj_ref, wi_ref, bi_ref, wo_ref, out_ref):
    eat = eat_ref[...]
    scal = jnp.maximum(
        lax.dot_general(eat, wi_ref[...], (((0,), (0,)), ((), ())),
                        preferred_element_type=jnp.float32)
        + bi_ref[...], 0.0)
    sel = (lax.broadcasted_iota(jnp.int32, (8, 1), 0) == 6
           ).astype(jnp.bfloat16)
    na_col = lax.dot_general(eat, sel, (((0,), (0,)), ((), ())),
                             preferred_element_type=jnp.float32)
    xj = xj_ref[...]
    base = jnp.concatenate(
        [xj, xj, xj, jnp.broadcast_to(na_col, (TE, D))], axis=1)
    msg = (scal * base).astype(jnp.bfloat16)
    out_ref[...] = jnp.dot(msg, wo_ref[...],
                           preferred_element_type=jnp.float32)


def _tc_mid(eat, xj, wi, bi, wo):
    return pl.pallas_call(
        _tc_mid_body,
        grid=(E // TE,),
        in_specs=[
            pl.BlockSpec((8, TE), lambda i: (0, i)),
            pl.BlockSpec((TE, D), lambda i: (i, 0)),
            pl.BlockSpec((8, KP2), lambda i: (0, 0)),
            pl.BlockSpec((1, KP2), lambda i: (0, 0)),
            pl.BlockSpec((KP2, OUT), lambda i: (0, 0)),
        ],
        out_specs=pl.BlockSpec((TE, OUT), lambda i: (i, 0)),
        out_shape=jax.ShapeDtypeStruct((E, OUT), jnp.float32),
    )(eat, xj, wi, bi, wo)


TN = 2000


def _tc_final_body(p0_ref, p1_ref, b_ref, out_ref):
    out_ref[...] = jnp.tanh(p0_ref[...] + p1_ref[...] + b_ref[...])


def _tc_final(p0, p1, b):
    return pl.pallas_call(
        _tc_final_body,
        grid=(N // TN,),
        in_specs=[
            pl.BlockSpec((TN, OUT), lambda i: (i, 0)),
            pl.BlockSpec((TN, OUT), lambda i: (i, 0)),
            pl.BlockSpec((1, OUT), lambda i: (0, 0)),
        ],
        out_specs=pl.BlockSpec((TN, OUT), lambda i: (i, 0)),
        out_shape=jax.ShapeDtypeStruct((N, OUT), jnp.float32),
    )(p0, p1, b)


_k2 = np.arange(KP2)
_VALID = _k2 < HID * IN_CH
_OLDK = np.where(_k2 < 384, (_k2 % D) * HID + _k2 // D,
                 np.minimum(_k2, HID * IN_CH - 1))


def kernel(x, edge_index, edge_attr, node_attr, W_in, b_in, W_out, b_out):
    src = edge_index[0]
    dst = edge_index[1]
    na = node_attr.reshape(N)
    wg = jnp.where(_VALID[:, None], W_in[_OLDK], 0.0)
    wi = jnp.concatenate(
        [wg.T, jnp.zeros((2, KP2), jnp.float32)], axis=0).astype(jnp.bfloat16)
    bi = jnp.where(_VALID, b_in[_OLDK], 0.0).reshape(1, KP2)
    wo = jnp.where(_VALID[:, None], W_out.T[_OLDK],
                   0.0).astype(jnp.bfloat16)

    xj, naj = _sc_gather(x, src, na)
    eat = jnp.concatenate(
        [edge_attr.T, naj.reshape(1, E), jnp.zeros((1, E), jnp.float32)],
        axis=0).astype(jnp.bfloat16)
    proj = _tc_mid(eat, xj, wi, bi, wo)
    parts = _sc_scatter(proj, dst)
    return _tc_final(parts[0], parts[1], b_out.reshape(1, OUT))
